# baseline (speedup 1.0000x reference)
import functools

import jax
import jax.numpy as jnp
from jax import lax
from jax.experimental import pallas as pl
from jax.experimental.pallas import tpu as pltpu
from jax.experimental.pallas import tpu_sc as plsc

VOCAB = 1000000
MAXLEN = 200
EMBED = 64
EMBED_PAD = 128
BATCH = 1024
SEQ = 200

LANES = 16
NW = 32
SEQ_PER_W = BATCH // NW
CHUNK = 104
HALF1 = SEQ - CHUNK
NBUF = 2
XCOLS = 128


def _make_kernel():
  mesh = plsc.VectorSubcoreMesh(core_axis_name="c", subcore_axis_name="s")

  @functools.partial(
      pl.kernel,
      mesh=mesh,
      compiler_params=pltpu.CompilerParams(needs_layout_passes=False),
      out_type=jax.ShapeDtypeStruct((BATCH, SEQ, EMBED), jnp.float32),
      scratch_types=[
          pltpu.VMEM((SEQ, XCOLS), jnp.int32),
          pltpu.VMEM((SEQ_PER_W, NBUF, CHUNK), jnp.int32),
          pltpu.VMEM((MAXLEN, EMBED), jnp.float32),
          [pltpu.VMEM((CHUNK, EMBED_PAD), jnp.float32) for _ in range(NBUF)],
          [pltpu.VMEM((CHUNK, EMBED), jnp.float32) for _ in range(NBUF)],
          [pltpu.SemaphoreType.DMA for _ in range(NBUF)],
          [pltpu.SemaphoreType.DMA for _ in range(NBUF)],
      ],
  )
  def embed(xt_hbm, tok_hbm, pos_hbm, out_hbm,
            slab_v, idx_v, pos_v, rows, outb, gsem, ssem):
    wid = lax.axis_index("s") * 2 + lax.axis_index("c")
    seq_base = wid * SEQ_PER_W
    group = wid // 4
    pltpu.sync_copy(xt_hbm.at[:, pl.ds(group * XCOLS, XCOLS)], slab_v)
    pltpu.sync_copy(pos_hbm, pos_v)

    lane = lax.iota(jnp.int32, LANES)
    zeros16 = jnp.full((LANES,), 0, jnp.int32)
    col0 = (wid - group * 4) * SEQ_PER_W

    def zero_body(sq, carry):
      idx_v[sq, 1, pl.ds(88, LANES)] = zeros16
      return carry

    lax.fori_loop(0, SEQ_PER_W, zero_body, 0)

    def trans_body(s, carry):
      half = lax.select(s < CHUNK, 0, 1)
      slot = zeros16 + (s - half * CHUNK)
      halfv = zeros16 + half
      for c0 in range(0, SEQ_PER_W, LANES):
        vals = slab_v[s, pl.ds(col0 + c0, LANES)]
        plsc.store_scatter(idx_v, [lane + c0, halfv, slot], vals)
      return carry

    lax.fori_loop(0, SEQ, trans_body, 0)
    plsc.subcore_barrier()

    def gather_start(b, seq):
      pltpu.async_copy(tok_hbm.at[idx_v.at[seq, b]], rows[b], gsem[b])

    def gather_wait(b):
      pltpu.make_async_copy(tok_hbm.at[pl.ds(0, CHUNK)], rows[b],
                            gsem[b]).wait()

    def _store_slices(b, seq):
      ln = CHUNK if b == 0 else HALF1
      src = outb[b].at[pl.ds(0, ln), :]
      dst = out_hbm.at[seq_base + seq, pl.ds(b * CHUNK, ln), :]
      return src, dst

    def store_start(b, seq):
      src, dst = _store_slices(b, seq)
      pltpu.async_copy(src, dst, ssem[b])

    def store_wait(b, seq):
      src, dst = _store_slices(b, seq)
      pltpu.make_async_copy(src, dst, ssem[b]).wait()

    def add_pos(b):
      ln = CHUNK if b == 0 else HALF1

      def row_body(r, rcarry):
        for d in range(EMBED // LANES):
          sl = pl.ds(d * LANES, LANES)
          outb[b][r, sl] = rows[b][r, sl] + pos_v[b * CHUNK + r, sl]
        return rcarry

      lax.fori_loop(0, ln, row_body, 0, unroll=4)

    for b in range(NBUF):
      gather_start(b, 0)

    def round_body(seq, carry):
      for b in range(NBUF):
        gather_wait(b)
        add_pos(b)
        store_start(b, seq)
      for b in range(NBUF):
        @pl.when(seq + 1 < SEQ_PER_W)
        def _():
          store_wait(b, seq)
          gather_start(b, seq + 1)

      return carry

    lax.fori_loop(0, SEQ_PER_W, round_body, 0)
    for b in range(NBUF):
      store_wait(b, SEQ_PER_W - 1)

  return embed


_embed = _make_kernel()


def kernel(x, token_table, pos_table):
  xt = jnp.swapaxes(x, 0, 1).astype(jnp.int32)
  tok_pad = jnp.pad(token_table, ((0, 0), (0, EMBED_PAD - EMBED)))
  return _embed(xt, tok_pad, pos_table)


# speedup vs baseline: 1.4363x; 1.4363x over previous
import functools

import jax
import jax.numpy as jnp
from jax import lax
from jax.experimental import pallas as pl
from jax.experimental.pallas import tpu as pltpu
from jax.experimental.pallas import tpu_sc as plsc

VOCAB = 1000000
MAXLEN = 200
EMBED = 64
BATCH = 1024
SEQ = 200

LANES = 16
NW = 32
SEQ_PER_W = BATCH // NW
CHUNK0 = 128
CHUNK1 = SEQ - CHUNK0
NBUF = 4
N_CHUNKS = 2 * SEQ_PER_W
N_ROUNDS = N_CHUNKS // NBUF


def _chunk_geom(c):
  return c // 2, (c % 2) * CHUNK0, CHUNK1 if c % 2 else CHUNK0


def _make_kernel():
  mesh = plsc.VectorSubcoreMesh(core_axis_name="c", subcore_axis_name="s")

  @functools.partial(
      pl.kernel,
      mesh=mesh,
      compiler_params=pltpu.CompilerParams(use_tc_tiling_on_sc=False),
      out_type=jax.ShapeDtypeStruct((BATCH, SEQ, EMBED), jnp.float32),
      scratch_types=[
          pltpu.VMEM((SEQ_PER_W, SEQ), jnp.int32),
          pltpu.VMEM((MAXLEN, EMBED), jnp.float32),
          [pltpu.VMEM((CHUNK1 if b % 2 else CHUNK0, EMBED), jnp.float32)
           for b in range(NBUF)],
          [pltpu.SemaphoreType.DMA for _ in range(NBUF)],
          [pltpu.SemaphoreType.DMA for _ in range(NBUF)],
      ],
  )
  def embed(x_hbm, tok_hbm, pos_hbm, out_hbm, idx_v, pos_v, rows, gsem, ssem):
    wid = lax.axis_index("s") * 2 + lax.axis_index("c")
    seq_base = wid * SEQ_PER_W
    pltpu.sync_copy(x_hbm.at[pl.ds(seq_base, SEQ_PER_W), :], idx_v)
    pltpu.sync_copy(pos_hbm, pos_v)

    def gather_start(b, seq, s0, ln):
      pltpu.async_copy(
          tok_hbm.at[idx_v.at[seq, pl.ds(s0, ln)]], rows[b], gsem[b]
      )

    def gather_wait(b, ln):
      pltpu.make_async_copy(tok_hbm.at[pl.ds(0, ln)], rows[b], gsem[b]).wait()

    def store_start(b, seq, s0, ln):
      pltpu.async_copy(
          rows[b], out_hbm.at[seq_base + seq, pl.ds(s0, ln), :], ssem[b]
      )

    def store_wait(b, seq, s0, ln):
      pltpu.make_async_copy(
          rows[b], out_hbm.at[seq_base + seq, pl.ds(s0, ln), :], ssem[b]
      ).wait()

    def add_pos(b, s0, ln):
      def row_body(r, rcarry):
        for d in range(EMBED // LANES):
          sl = pl.ds(d * LANES, LANES)
          plsc.addupdate(rows[b].at[r, sl], pos_v[s0 + r, sl])
        return rcarry

      lax.fori_loop(0, ln, row_body, 0, unroll=4)

    for b in range(NBUF):
      seq, s0, ln = _chunk_geom(b)
      gather_start(b, seq, s0, ln)

    def round_body(i, carry):
      c0 = i * NBUF
      for b in range(NBUF):
        s0 = (b % 2) * CHUNK0
        ln = CHUNK1 if b % 2 else CHUNK0
        seq = (c0 + b) // 2
        gather_wait(b, ln)
        add_pos(b, s0, ln)
        store_start(b, seq, s0, ln)
      for b in range(NBUF):
        s0 = (b % 2) * CHUNK0
        ln = CHUNK1 if b % 2 else CHUNK0
        seq = (c0 + b) // 2

        @pl.when(c0 + b + NBUF < N_CHUNKS)
        def _():
          store_wait(b, seq, s0, ln)
          gather_start(b, (c0 + b + NBUF) // 2, s0, ln)

      return carry

    lax.fori_loop(0, N_ROUNDS, round_body, 0)
    for b in range(NBUF):
      seq, s0, ln = _chunk_geom(N_CHUNKS - NBUF + b)
      store_wait(b, seq, s0, ln)

  return embed


_embed = _make_kernel()


def kernel(x, token_table, pos_table):
  return _embed(x.astype(jnp.int32), token_table, pos_table)
